# Initial kernel scaffold; baseline (speedup 1.0000x reference)
import jax
import jax.numpy as jnp
from jax.experimental import pallas as pl


def kernel(x, psi_ker_idx, psi_row_idx, psi_col_idx, psi_vals, weight, bias):
    raise NotImplementedError("write your pallas kernel here")


# speedup vs baseline: 12.8482x; 12.8482x over previous
import jax
import jax.numpy as jnp
from jax import lax
from jax.experimental import pallas as pl
from jax.experimental.pallas import tpu as pltpu
from jax.experimental.pallas import tpu_sc as plsc

NC, NS, L = 2, 16, 16
NW = NC * NS
K = 3
WO = 180
WP = 192
NACC = 12
NCHUNK = WP // NACC
WPAD = 384
NLAT_W = 5
BC_CH = 16


def _sc_sparse_stage(xpad5, off, val, ptr2d, BCJ, H, HO):
    S16P = off.shape[0]
    NJOBS = HO * BCJ
    NJ = -(-NJOBS // NW)

    mesh = plsc.VectorSubcoreMesh(core_axis_name="c", subcore_axis_name="s",
                                  num_cores=NC, num_subcores=NS)

    def body(xpad_hbm, off_hbm, val_hbm, ptr_hbm, y_hbm,
             off_v, val_v, ptr_v, xwin_v, stage_v):
        wid = lax.axis_index("s") * NC + lax.axis_index("c")
        pltpu.sync_copy(off_hbm, off_v)
        pltpu.sync_copy(val_hbm, val_v)
        pltpu.sync_copy(ptr_hbm, ptr_v)

        def job_body(n, carry):
            j = n * NW + wid

            @pl.when(j < NJOBS)
            def _():
                i = j // BCJ
                cb = j % BCJ
                lat0 = jnp.clip(2 * i - 2, 0, H - NLAT_W)
                pltpu.sync_copy(xpad_hbm.at[cb, pl.ds(2 * lat0, 2 * NLAT_W)],
                                xwin_v)
                pr = ptr_v[i]

                def chunk(pc, carry2):
                    po0 = pc * NACC
                    for k in range(K):
                        t0 = pr[k]
                        nb = (pr[k + 1] - t0) >> 4

                        def blk(b, acc):
                            base = t0 + b * L
                            offv = off_v[pl.ds(base, L)]
                            valv = val_v[pl.ds(base, L)]
                            for li in range(L):
                                o = offv[li]
                                v = valv[li]
                                ab = o >> 9
                                mp = (o & 511) + po0
                                acc = tuple(
                                    acc[wv] + v * xwin_v[ab, mp + wv, :]
                                    for wv in range(NACC))
                            return acc

                        acc0 = tuple(jnp.zeros((L,), jnp.float32)
                                     for _ in range(NACC))
                        acc = lax.fori_loop(0, nb, blk, acc0)
                        for wv in range(NACC):
                            stage_v[k, po0 + wv] = acc[wv]
                    return carry2

                lax.fori_loop(0, NCHUNK, chunk, 0)
                pltpu.sync_copy(stage_v, y_hbm.at[cb, i])
            return carry

        lax.fori_loop(0, NJ, job_body, 0)

    fn = pl.kernel(
        body,
        out_type=jax.ShapeDtypeStruct((BCJ, HO, K, WP, BC_CH), jnp.float32),
        mesh=mesh,
        compiler_params=pltpu.CompilerParams(use_tc_tiling_on_sc=False),
        scratch_types=[
            pltpu.VMEM((S16P,), jnp.int32),
            pltpu.VMEM((S16P,), jnp.float32),
            pltpu.VMEM((HO, L), jnp.int32),
            pltpu.VMEM((2 * NLAT_W, WPAD, BC_CH), jnp.float32),
            pltpu.VMEM((K, WP, BC_CH), jnp.float32),
        ],
    )
    return fn(xpad5, off, val, ptr2d)


def _tc_einsum_body(w2_ref, y_ref, bias_ref, out_ref):
    res = lax.dot_general(w2_ref[...], y_ref[0],
                          dimension_numbers=(((1,), (0,)), ((), ())),
                          preferred_element_type=jnp.float32)
    out_ref[0] = res + bias_ref[...]


def _tc_einsum(w2, y3, bias2, B, O, CK, NCOL):
    return pl.pallas_call(
        _tc_einsum_body,
        grid=(B,),
        in_specs=[
            pl.BlockSpec((O, CK), lambda b: (0, 0)),
            pl.BlockSpec((1, CK, NCOL), lambda b: (b, 0, 0)),
            pl.BlockSpec((O, 1), lambda b: (0, 0)),
        ],
        out_specs=pl.BlockSpec((1, O, NCOL), lambda b: (b, 0, 0)),
        out_shape=jax.ShapeDtypeStruct((B, O, NCOL), jnp.float32),
    )(w2, y3, bias2)


def kernel(x, psi_ker_idx, psi_row_idx, psi_col_idx, psi_vals, weight, bias):
    B, C, H, W = x.shape
    BC = B * C
    BCJ = BC // BC_CH
    HO = (H + 1) // 2
    O = weight.shape[0]
    NNZ = psi_vals.shape[0]
    NSEG = K * HO
    S16P = -(-(NNZ + NSEG * (L - 1)) // L) * L

    xb = x.reshape(BC, H, WO, 2).transpose(0, 1, 3, 2)
    xpad = jnp.concatenate([xb, xb, xb[..., : WPAD - 2 * WO]], axis=-1)
    xpad5 = (xpad.reshape(BCJ, BC_CH, H * 2, WPAD)
                 .transpose(0, 2, 3, 1))

    lat = psi_col_idx // W
    lon = psi_col_idx % W
    r = lon % 2
    m = lon // 2
    lat0 = jnp.clip(2 * psi_row_idx - 2, 0, H - NLAT_W)
    a = lat - lat0
    off = (((a * 2 + r) << 9) + m).astype(jnp.int32)
    key = (psi_row_idx * K + psi_ker_idx).astype(jnp.int32)
    ptr = jnp.searchsorted(key, jnp.arange(NSEG + 1, dtype=jnp.int32),
                           side="left").astype(jnp.int32)
    nseg = ptr[1:] - ptr[:-1]
    seg16 = -(-nseg // L) * L
    starts16 = jnp.concatenate(
        [jnp.zeros((1,), jnp.int32), jnp.cumsum(seg16).astype(jnp.int32)])
    dst = starts16[key] + (jnp.arange(NNZ, dtype=jnp.int32) - ptr[key])
    offp = jnp.zeros((S16P,), jnp.int32).at[dst].set(off)
    valp = jnp.zeros((S16P,), jnp.float32).at[dst].set(psi_vals)
    ptr2d = jnp.zeros((HO, L), jnp.int32)
    rows4 = (jnp.arange(HO, dtype=jnp.int32)[:, None] * K
             + jnp.arange(K + 1, dtype=jnp.int32)[None, :])
    ptr2d = ptr2d.at[:, : K + 1].set(starts16[rows4])

    y = _sc_sparse_stage(xpad5, offp, valp, ptr2d, BCJ, H, HO)

    w2 = weight.reshape(O, -1)
    CK = w2.shape[1]
    y3 = (y.transpose(0, 4, 2, 1, 3)
           .reshape(B, C * K, HO * WP))
    out = _tc_einsum(w2, y3, bias.reshape(O, 1), B, O, CK, HO * WP)
    return out.reshape(B, O, HO, WP)[..., :WO]
